# baseline (speedup 1.0000x reference)
import functools

import jax
import jax.numpy as jnp
from jax import lax
from jax.experimental import pallas as pl
from jax.experimental.pallas import tpu as pltpu
from jax.experimental.pallas import tpu_sc as plsc

_NCLS = 19
_ROWP = 32
_HROWS = 20
_HSIZE = _HROWS * _ROWP
_STRIDE = _HSIZE + 1
_LANES = 16
_NWORKERS = 32
_RBLK = 128


def _argmax_bin_body(pref, tref, oref):
    x = pref[0]
    best = x[0]
    pc = jnp.zeros(best.shape, jnp.int32)
    for c in range(1, _NCLS):
        v = x[c]
        take = v > best
        best = jnp.where(take, v, best)
        pc = jnp.where(take, c, pc)
    t = tref[0]
    valid = (t >= 0) & (t < _NCLS)
    binv = jnp.where(valid, t * _ROWP + pc, _NCLS * _ROWP)
    ci = lax.broadcasted_iota(jnp.int32, binv.shape, 1)
    oref[0] = binv + (ci % _LANES) * _STRIDE


def _stage1(preds, target):
    b, c, h, w = preds.shape
    nch = h // _RBLK
    return pl.pallas_call(
        _argmax_bin_body,
        grid=(b, nch),
        in_specs=[
            pl.BlockSpec((1, c, _RBLK, w), lambda i, j: (i, 0, j, 0)),
            pl.BlockSpec((1, _RBLK, w), lambda i, j: (i, j, 0)),
        ],
        out_specs=pl.BlockSpec((1, _RBLK, w), lambda i, j: (i, j, 0)),
        out_shape=jax.ShapeDtypeStruct((b, h, w), jnp.int32),
    )(preds, target)


def _make_sc_hist(b, h, w):
    rows_per_w = (b * h) // _NWORKERS
    vecs_per_row = w // _LANES
    mesh = plsc.VectorSubcoreMesh(core_axis_name="c", subcore_axis_name="s")

    @functools.partial(
        pl.kernel,
        mesh=mesh,
        compiler_params=pltpu.CompilerParams(needs_layout_passes=False),
        out_type=jax.ShapeDtypeStruct((_NWORKERS, _HROWS, _ROWP), jnp.float32),
        scratch_types=[
            pltpu.VMEM((rows_per_w, w), jnp.int32),
            pltpu.VMEM((((_LANES * _STRIDE + 127) // 128) * 128,), jnp.float32),
            pltpu.VMEM((_HROWS, _ROWP), jnp.float32),
        ],
    )
    def sc_hist(bins_hbm, out_hbm, idx_v, hist_v, red_v):
        wid = lax.axis_index("s") * 2 + lax.axis_index("c")
        wpb = h // rows_per_w
        bi = wid // wpb
        r0 = (wid % wpb) * rows_per_w
        pltpu.sync_copy(bins_hbm.at[bi, pl.ds(r0, rows_per_w), :], idx_v)

        zero16 = jnp.zeros((_LANES,), jnp.float32)

        def zero_body(i, carry):
            for u in range(8):
                hist_v[pl.ds((i * 8 + u) * _LANES, _LANES)] = zero16
            return carry

        lax.fori_loop(0, _LANES * _STRIDE // _LANES // 8 + 1, zero_body, 0)

        ones = jnp.ones((_LANES,), jnp.float32)

        @plsc.parallel_loop(0, rows_per_w, unroll=2)
        def _scat(r):
            for u in range(vecs_per_row):
                a = idx_v[r, pl.ds(u * _LANES, _LANES)]
                plsc.addupdate_scatter(hist_v, [a], ones)

        nslots = _HSIZE // _LANES
        accs = [hist_v[pl.ds(j * _LANES, _LANES)] for j in range(nslots)]
        for r in range(1, _LANES):
            base = r * _STRIDE
            vals = [hist_v[pl.ds(base + j * _LANES, _LANES)] for j in range(nslots)]
            accs = [a + v for a, v in zip(accs, vals)]
        for j in range(nslots):
            red_v[j // 2, pl.ds((j % 2) * _LANES, _LANES)] = accs[j]
        pltpu.sync_copy(red_v, out_hbm.at[wid])

    return sc_hist


def _iou_body(href, oref):
    h = href[...]
    hs = jnp.sum(h, axis=0)
    ri = lax.broadcasted_iota(jnp.int32, (_HROWS, _ROWP), 0)
    ci = lax.broadcasted_iota(jnp.int32, (_HROWS, _ROWP), 1)
    rowsum = jnp.sum(hs, axis=1, keepdims=True)
    colsum = jnp.sum(jnp.where(ri < _NCLS, hs, 0.0), axis=0, keepdims=True)
    union = rowsum + colsum - hs
    iou = hs / jnp.maximum(union, 1.0)
    oref[...] = jnp.sum(jnp.where(ri == ci, iou, 0.0), axis=0, keepdims=True)


def _stage3(part):
    return pl.pallas_call(
        _iou_body,
        out_shape=jax.ShapeDtypeStruct((1, _ROWP), jnp.float32),
    )(part)


def kernel(preds, target):
    b, c, h, w = preds.shape
    bins = _stage1(preds, target)
    part = _make_sc_hist(b, h, w)(bins)
    ious = _stage3(part)
    return ious[0, :_NCLS]


# speedup vs baseline: 1.0462x; 1.0462x over previous
import functools

import jax
import jax.numpy as jnp
from jax import lax
from jax.experimental import pallas as pl
from jax.experimental.pallas import tpu as pltpu
from jax.experimental.pallas import tpu_sc as plsc

_NCLS = 19
_ROWP = 32
_HROWS = 20
_HSIZE = _HROWS * _ROWP
_STRIDE = _HSIZE + 1
_LANES = 16
_NWORKERS = 32
_RBLK = 128


def _argmax_bin_body(pref, tref, oref):
    x = pref[0]
    best = x[0]
    pc = jnp.zeros(best.shape, jnp.int32)
    for c in range(1, _NCLS):
        v = x[c]
        take = v > best
        best = jnp.where(take, v, best)
        pc = jnp.where(take, c, pc)
    t = tref[0]
    valid = (t >= 0) & (t < _NCLS)
    binv = jnp.where(valid, t * _ROWP + pc, _NCLS * _ROWP)
    ci = lax.broadcasted_iota(jnp.int32, binv.shape, 1)
    oref[0] = (binv + ((ci // 2) % _LANES) * _STRIDE).astype(jnp.int16)


def _stage1(preds, target):
    b, c, h, w = preds.shape
    nch = h // _RBLK
    return pl.pallas_call(
        _argmax_bin_body,
        grid=(b, nch),
        in_specs=[
            pl.BlockSpec((1, c, _RBLK, w), lambda i, j: (i, 0, j, 0)),
            pl.BlockSpec((1, _RBLK, w), lambda i, j: (i, j, 0)),
        ],
        out_specs=pl.BlockSpec((1, _RBLK, w), lambda i, j: (i, j, 0)),
        out_shape=jax.ShapeDtypeStruct((b, h, w), jnp.int16),
    )(preds, target)


def _make_sc_hist(b, h, w):
    rows_per_w = (b * h) // _NWORKERS
    vecs_per_row = w // _LANES
    mesh = plsc.VectorSubcoreMesh(core_axis_name="c", subcore_axis_name="s")

    @functools.partial(
        pl.kernel,
        mesh=mesh,
        compiler_params=pltpu.CompilerParams(needs_layout_passes=False),
        out_type=jax.ShapeDtypeStruct((_NWORKERS, _HROWS, _ROWP), jnp.float32),
        scratch_types=[
            pltpu.VMEM((rows_per_w, w), jnp.int16),
            pltpu.SemaphoreType.DMA,
            pltpu.VMEM((((_LANES * _STRIDE + 127) // 128) * 128,), jnp.float32),
            pltpu.VMEM((_HROWS, _ROWP), jnp.float32),
        ],
    )
    def sc_hist(bins_hbm, out_hbm, idx_v, sem, hist_v, red_v):
        wid = lax.axis_index("s") * 2 + lax.axis_index("c")
        wpb = h // rows_per_w
        bi = wid // wpb
        r0 = (wid % wpb) * rows_per_w
        cp = pltpu.async_copy(bins_hbm.at[bi, pl.ds(r0, rows_per_w), :], idx_v, sem)

        zero16 = jnp.zeros((_LANES,), jnp.float32)

        def zero_body(i, carry):
            for u in range(8):
                hist_v[pl.ds((i * 8 + u) * _LANES, _LANES)] = zero16
            return carry

        lax.fori_loop(0, _LANES * _STRIDE // _LANES // 8 + 1, zero_body, 0)
        cp.wait()

        ones = jnp.ones((_LANES,), jnp.float32)

        n_pairs = w // 32

        @plsc.parallel_loop(0, rows_per_w, unroll=1)
        def _scat(r):
            packed = [idx_v[r, pl.ds(u * 32, 32)] for u in range(n_pairs)]
            halves = [
                plsc.unpack(pk, format=plsc.PackFormat.INTERLEAVED)
                for pk in packed
            ]
            for a, b2 in halves:
                plsc.addupdate_scatter(hist_v, [a], ones)
                plsc.addupdate_scatter(hist_v, [b2], ones)

        nslots = _HSIZE // _LANES
        accs = [hist_v[pl.ds(j * _LANES, _LANES)] for j in range(nslots)]
        for r in range(1, _LANES):
            base = r * _STRIDE
            vals = [hist_v[pl.ds(base + j * _LANES, _LANES)] for j in range(nslots)]
            accs = [a + v for a, v in zip(accs, vals)]
        for j in range(nslots):
            red_v[j // 2, pl.ds((j % 2) * _LANES, _LANES)] = accs[j]
        pltpu.sync_copy(red_v, out_hbm.at[wid])

    return sc_hist


def _iou_body(href, oref):
    h = href[...]
    hs = jnp.sum(h, axis=0)
    ri = lax.broadcasted_iota(jnp.int32, (_HROWS, _ROWP), 0)
    ci = lax.broadcasted_iota(jnp.int32, (_HROWS, _ROWP), 1)
    rowsum = jnp.sum(hs, axis=1, keepdims=True)
    colsum = jnp.sum(jnp.where(ri < _NCLS, hs, 0.0), axis=0, keepdims=True)
    union = rowsum + colsum - hs
    iou = hs / jnp.maximum(union, 1.0)
    oref[...] = jnp.sum(jnp.where(ri == ci, iou, 0.0), axis=0, keepdims=True)


def _stage3(part):
    return pl.pallas_call(
        _iou_body,
        out_shape=jax.ShapeDtypeStruct((1, _ROWP), jnp.float32),
    )(part)


def kernel(preds, target):
    b, c, h, w = preds.shape
    bins = _stage1(preds, target)
    part = _make_sc_hist(b, h, w)(bins)
    ious = _stage3(part)
    return ious[0, :_NCLS]
